# Initial kernel scaffold; baseline (speedup 1.0000x reference)
import jax
import jax.numpy as jnp
from jax.experimental import pallas as pl


def kernel(x, edge_index, D1, edge_len, batch, wp1_W, wp1_b, W1_0, W1_1, wp2_W, wp2_b, W2_0, W2_1, wp3_W, wp3_b, W3_0, Wout, bout):
    raise NotImplementedError("write your pallas kernel here")


# speedup vs baseline: 84.6035x; 84.6035x over previous
import functools

import jax
import jax.numpy as jnp
import numpy as np
from jax import lax
from jax.experimental import pallas as pl
from jax.experimental.pallas import tpu as pltpu
from jax.experimental.pallas import tpu_sc as plsc

H = 4
G = 64
NCORES = 2
NSUB = 16
NW = NCORES * NSUB
SUB = 80
MU = [float(v) for v in np.linspace(0.7, 1.7, 10)]
LOG2 = float(np.log(2.0))


def _iota16():
    return lax.broadcasted_iota(jnp.int32, (16,), 0)


def _emb_vecs(el):
    return [jnp.exp((el - m) * (el - m) * -50.0) for m in MU]


def _cutoff(el):
    u = el * (1.0 / 1.5)
    u2 = u * u
    u4 = u2 * u2
    u6 = u4 * u2
    u7 = u6 * u
    u8 = u4 * u4
    cut = 1.0 - 28.0 * u6 + 48.0 * u7 - 21.0 * u8
    return jnp.where(u < 1.0, cut, jnp.zeros_like(cut))


def _lw_vecs(embs, wp_ref, wpb_ref, nout):
    out = []
    for o in range(nout):
        v = embs[0] * wp_ref[0 * nout + o, :]
        for k in range(1, 10):
            v = v + embs[k] * wp_ref[k * nout + o, :]
        out.append(v + wpb_ref[o, :])
    return out


def _sc_layer1(x1, src2, dst2, d1r, elen, wpW, wpb, z16, N, E):
    K = 400
    NCH = (E // NW) // K
    NSB = K // SUB
    mesh = plsc.VectorSubcoreMesh(core_axis_name="c", subcore_axis_name="s")

    @functools.partial(
        pl.kernel,
        out_type=jax.ShapeDtypeStruct((2 * N, 16), jnp.float32),
        mesh=mesh,
        compiler_params=pltpu.CompilerParams(
            needs_layout_passes=False, use_tc_tiling_on_sc=False),
        scratch_types=[
            pltpu.VMEM((N,), jnp.float32),
            pltpu.VMEM((K,), jnp.int32),
            pltpu.VMEM((NSB, SUB), jnp.int32),
            pltpu.VMEM((K, 16), jnp.float32),
            pltpu.VMEM((K,), jnp.float32),
            pltpu.VMEM((K, 16), jnp.float32),
            pltpu.VMEM((80, 16), jnp.float32),
            pltpu.VMEM((8, 16), jnp.float32),
            pltpu.VMEM_SHARED((N, 16), jnp.float32),
        ],
    )
    def k(x_hbm, src_hbm, dst_hbm, d1_hbm, el_hbm, wpW_hbm, wpb_hbm, z_hbm,
          out_hbm, xv, srcv, dstv, d1v, elv, mrows, wpv, wpbv, acc):
        cid = lax.axis_index("c")
        sid = lax.axis_index("s")
        wid = sid * NCORES + cid

        if True:
            @pl.when(sid == 0)
            def _zero_acc():
                pltpu.sync_copy(z_hbm, acc)
            pltpu.sync_copy(x_hbm, xv)
            pltpu.sync_copy(wpW_hbm, wpv)
            pltpu.sync_copy(wpb_hbm, wpbv)
            plsc.subcore_barrier()

            ebase = wid * (E // NW)

            def chunk(ci, carry):
                off = ebase + ci * K
                pltpu.sync_copy(src_hbm.at[pl.ds(off, K)], srcv)
                for s0 in range(NSB):
                    pltpu.sync_copy(dst_hbm.at[pl.ds(off + s0 * SUB, SUB)],
                                    dstv.at[s0])
                pltpu.sync_copy(d1_hbm.at[pl.ds(off, K)], d1v)
                pltpu.sync_copy(el_hbm.at[pl.ds(off, K)], elv)

                def sub(s, c2):
                    def grp(gg, c3):
                        b = s * SUB + gg * 16
                        rows16 = _iota16() + b
                        el = elv[pl.ds(b, 16)]
                        src16 = srcv[pl.ds(b, 16)]
                        xs = plsc.load_gather(xv, [src16])
                        d0 = [plsc.load_gather(
                            d1v, [rows16, jnp.full((16,), j, jnp.int32)])
                            for j in range(4)]
                        d2 = [plsc.load_gather(
                            d1v, [rows16, jnp.full((16,), 8 + j, jnp.int32)])
                            for j in range(4)]
                        embs = _emb_vecs(el)
                        lw = _lw_vecs(embs, wpv, wpbv, 8)
                        cut = _cutoff(el)
                        xr = d0[0] * xs
                        y0 = [xr * lw[c] for c in range(H)]
                        y2 = [xr * lw[4 + c] for c in range(H)]
                        for i in range(4):
                            for c in range(H):
                                o = (d0[i] * y0[c] + d2[i] * y2[c]) * cut
                                plsc.store_scatter(
                                    mrows,
                                    [rows16,
                                     jnp.full((16,), i * 4 + c, jnp.int32)],
                                    o)
                        return c3
                    lax.fori_loop(0, SUB // 16, grp, c2)
                    return c2
                lax.fori_loop(0, NSB, sub, 0)
                for s in range(NSB):
                    pltpu.sync_copy(mrows.at[pl.ds(s * SUB, SUB)],
                                    acc.at[dstv.at[s]], add=True)
                return carry
            lax.fori_loop(0, NCH, chunk, 0)
            plsc.subcore_barrier()

            @pl.when(sid == 0)
            def _readout():
                pltpu.sync_copy(acc, out_hbm.at[pl.ds(cid * N, N)])

    return k(x1, src2, dst2, d1r, elen, wpW, wpb, z16)


def _sc_layer2(h, src2, dst2, d1r, elen, wpW, wpb, z16, N, E):
    K = 400
    NCH = (E // NW) // K
    NSB = K // SUB
    mesh = plsc.VectorSubcoreMesh(core_axis_name="c", subcore_axis_name="s")

    @functools.partial(
        pl.kernel,
        out_type=jax.ShapeDtypeStruct((2 * N, 16), jnp.float32),
        mesh=mesh,
        compiler_params=pltpu.CompilerParams(
            needs_layout_passes=False, use_tc_tiling_on_sc=False),
        scratch_types=[
            pltpu.VMEM((K,), jnp.int32),
            pltpu.VMEM((NSB, SUB), jnp.int32),
            pltpu.VMEM((K, 16), jnp.float32),
            pltpu.VMEM((K,), jnp.float32),
            pltpu.VMEM((K, 16), jnp.float32),
            pltpu.VMEM((K, 16), jnp.float32),
            pltpu.VMEM((240, 16), jnp.float32),
            pltpu.VMEM((24, 16), jnp.float32),
            pltpu.VMEM_SHARED((N, 16), jnp.float32),
            pltpu.VMEM_SHARED((N, 16), jnp.float32),
            pltpu.SemaphoreType.DMA,
        ],
    )
    def k(h_hbm, src_hbm, dst_hbm, d1_hbm, el_hbm, wpW_hbm, wpb_hbm, z_hbm,
          out_hbm, srcv, dstv, d1v, elv, hrows, mrows, wpv, wpbv, htab, acc,
          sem):
        cid = lax.axis_index("c")
        sid = lax.axis_index("s")
        wid = sid * NCORES + cid

        if True:
            @pl.when(sid == 0)
            def _zero_acc():
                pltpu.sync_copy(z_hbm, acc)

            @pl.when(sid == 1)
            def _stage_h():
                pltpu.sync_copy(h_hbm, htab)
            pltpu.sync_copy(wpW_hbm, wpv)
            pltpu.sync_copy(wpb_hbm, wpbv)
            plsc.subcore_barrier()

            ebase = wid * (E // NW)

            def chunk(ci, carry):
                off = ebase + ci * K
                pltpu.sync_copy(src_hbm.at[pl.ds(off, K)], srcv)
                for s0 in range(NSB):
                    pltpu.sync_copy(dst_hbm.at[pl.ds(off + s0 * SUB, SUB)],
                                    dstv.at[s0])
                pltpu.sync_copy(d1_hbm.at[pl.ds(off, K)], d1v)
                pltpu.sync_copy(el_hbm.at[pl.ds(off, K)], elv)
                cps = [pltpu.async_copy(htab.at[srcv.at[pl.ds(s * SUB, SUB)]],
                                        hrows.at[pl.ds(s * SUB, SUB)], sem)
                       for s in range(NSB)]
                for cp in cps:
                    cp.wait()

                def sub(s, c2):
                    def grp(gg, c3):
                        b = s * SUB + gg * 16
                        rows16 = _iota16() + b
                        el = elv[pl.ds(b, 16)]
                        d = [plsc.load_gather(
                            d1v, [rows16, jnp.full((16,), j, jnp.int32)])
                            for j in range(16)]
                        hs = [plsc.load_gather(
                            hrows, [rows16, jnp.full((16,), j, jnp.int32)])
                            for j in range(16)]
                        embs = _emb_vecs(el)
                        lw = _lw_vecs(embs, wpv, wpbv, 24)
                        cut = _cutoff(el)
                        xj = [[None] * H for _ in range(4)]
                        for i in range(4):
                            for c in range(H):
                                v = d[i * 4] * hs[c]
                                for j in range(1, 4):
                                    v = v + d[i * 4 + j] * hs[j * 4 + c]
                                xj[i][c] = v
                        m = [[None] * H for _ in range(4)]
                        for c in range(H):
                            m[0][c] = lw[c] * xj[0][c] + lw[8 + c] * xj[2][c]
                            m[2][c] = lw[4 + c] * xj[0][c] + lw[12 + c] * xj[2][c]
                            m[3][c] = lw[16 + c] * xj[3][c] - lw[20 + c] * xj[1][c]
                            m[1][c] = lw[20 + c] * xj[3][c] + lw[16 + c] * xj[1][c]
                        for i in range(4):
                            for c in range(H):
                                v = d[i] * m[0][c]
                                for j in range(1, 4):
                                    v = v + d[j * 4 + i] * m[j][c]
                                plsc.store_scatter(
                                    mrows,
                                    [rows16,
                                     jnp.full((16,), i * 4 + c, jnp.int32)],
                                    v * cut)
                        return c3
                    lax.fori_loop(0, SUB // 16, grp, c2)
                    return c2
                lax.fori_loop(0, NSB, sub, 0)
                for s in range(NSB):
                    pltpu.sync_copy(mrows.at[pl.ds(s * SUB, SUB)],
                                    acc.at[dstv.at[s]], add=True)
                return carry
            lax.fori_loop(0, NCH, chunk, 0)
            plsc.subcore_barrier()

            @pl.when(sid == 0)
            def _readout():
                pltpu.sync_copy(acc, out_hbm.at[pl.ds(cid * N, N)])

    return k(h, src2, dst2, d1r, elen, wpW, wpb, z16)


def _sc_layer3(h, src2, dst2, d1r, elen, wpW, wpb, z8, N, E):
    K = 400
    NCH = (E // NW) // K
    NSB = K // SUB
    W3 = 8
    mesh = plsc.VectorSubcoreMesh(core_axis_name="c", subcore_axis_name="s")

    @functools.partial(
        pl.kernel,
        out_type=jax.ShapeDtypeStruct((2 * N, W3), jnp.float32),
        mesh=mesh,
        compiler_params=pltpu.CompilerParams(
            needs_layout_passes=False, use_tc_tiling_on_sc=False),
        scratch_types=[
            pltpu.VMEM((K,), jnp.int32),
            pltpu.VMEM((NSB, SUB), jnp.int32),
            pltpu.VMEM((K, 16), jnp.float32),
            pltpu.VMEM((K,), jnp.float32),
            pltpu.VMEM((K, 16), jnp.float32),
            pltpu.VMEM((K, W3), jnp.float32),
            pltpu.VMEM((80, 16), jnp.float32),
            pltpu.VMEM((8, 16), jnp.float32),
            pltpu.VMEM_SHARED((N, 16), jnp.float32),
            pltpu.VMEM_SHARED((N, W3), jnp.float32),
            pltpu.SemaphoreType.DMA,
        ],
    )
    def k(h_hbm, src_hbm, dst_hbm, d1_hbm, el_hbm, wpW_hbm, wpb_hbm, z_hbm,
          out_hbm, srcv, dstv, d1v, elv, hrows, mrows, wpv, wpbv, htab, acc,
          sem):
        cid = lax.axis_index("c")
        sid = lax.axis_index("s")
        wid = sid * NCORES + cid

        if True:
            iota = _iota16()
            zr = lax.shift_right_logical(iota, 3)
            zc = lax.bitwise_and(iota, 7)
            zv = jnp.zeros((16,), jnp.float32)

            def zq(i, c):
                plsc.store_scatter(mrows, [i * 2 + zr, zc], zv)
                return c
            lax.fori_loop(0, K // 2, zq, 0)

            @pl.when(sid == 0)
            def _zero_acc():
                pltpu.sync_copy(z_hbm, acc)

            @pl.when(sid == 1)
            def _stage_h():
                pltpu.sync_copy(h_hbm, htab)
            pltpu.sync_copy(wpW_hbm, wpv)
            pltpu.sync_copy(wpb_hbm, wpbv)
            plsc.subcore_barrier()

            ebase = wid * (E // NW)

            def chunk(ci, carry):
                off = ebase + ci * K
                pltpu.sync_copy(src_hbm.at[pl.ds(off, K)], srcv)
                for s0 in range(NSB):
                    pltpu.sync_copy(dst_hbm.at[pl.ds(off + s0 * SUB, SUB)],
                                    dstv.at[s0])
                pltpu.sync_copy(d1_hbm.at[pl.ds(off, K)], d1v)
                pltpu.sync_copy(el_hbm.at[pl.ds(off, K)], elv)
                cps = [pltpu.async_copy(htab.at[srcv.at[pl.ds(s * SUB, SUB)]],
                                        hrows.at[pl.ds(s * SUB, SUB)], sem)
                       for s in range(NSB)]
                for cp in cps:
                    cp.wait()

                def sub(s, c2):
                    def grp(gg, c3):
                        b = s * SUB + gg * 16
                        rows16 = _iota16() + b
                        el = elv[pl.ds(b, 16)]
                        d0 = [plsc.load_gather(
                            d1v, [rows16, jnp.full((16,), j, jnp.int32)])
                            for j in range(4)]
                        d2 = [plsc.load_gather(
                            d1v, [rows16, jnp.full((16,), 8 + j, jnp.int32)])
                            for j in range(4)]
                        hs = [plsc.load_gather(
                            hrows, [rows16, jnp.full((16,), j, jnp.int32)])
                            for j in range(16)]
                        embs = _emb_vecs(el)
                        lw = _lw_vecs(embs, wpv, wpbv, 8)
                        cut = _cutoff(el)
                        dc = d0[0] * cut
                        for c in range(H):
                            xj0 = d0[0] * hs[c]
                            xj2 = d2[0] * hs[c]
                            for j in range(1, 4):
                                xj0 = xj0 + d0[j] * hs[j * 4 + c]
                                xj2 = xj2 + d2[j] * hs[j * 4 + c]
                            y0 = lw[c] * xj0 + lw[4 + c] * xj2
                            plsc.store_scatter(
                                mrows,
                                [rows16, jnp.full((16,), c, jnp.int32)],
                                y0 * dc)
                        return c3
                    lax.fori_loop(0, SUB // 16, grp, c2)
                    return c2
                lax.fori_loop(0, NSB, sub, 0)
                for s in range(NSB):
                    pltpu.sync_copy(mrows.at[pl.ds(s * SUB, SUB)],
                                    acc.at[dstv.at[s]], add=True)
                return carry
            lax.fori_loop(0, NCH, chunk, 0)
            plsc.subcore_barrier()

            @pl.when(sid == 0)
            def _readout():
                pltpu.sync_copy(acc, out_hbm.at[pl.ds(cid * N, N)])

    return k(h, src2, dst2, d1r, elen, wpW, wpb, z8)


def _tc_node(parts, M, S, N):
    B = 2000
    NB = N // B

    def body(p_ref, m_ref, s_ref, o_ref):
        ph = p_ref[0] + p_ref[1]
        h2 = jnp.dot(ph, m_ref[...], preferred_element_type=jnp.float32)
        nsq = jnp.dot(h2 * h2, s_ref[...], preferred_element_type=jnp.float32)
        n = jnp.sqrt(nsq + 1e-12)
        act = jax.nn.softplus(n) - LOG2
        o_ref[...] = h2 * (act / n)

    return pl.pallas_call(
        body,
        grid=(NB,),
        in_specs=[
            pl.BlockSpec((2, B, 16), lambda i: (0, i, 0)),
            pl.BlockSpec((16, 16), lambda i: (0, 0)),
            pl.BlockSpec((16, 16), lambda i: (0, 0)),
        ],
        out_specs=pl.BlockSpec((B, 16), lambda i: (i, 0)),
        out_shape=jax.ShapeDtypeStruct((N, 16), jnp.float32),
    )(parts, M, S)


def _tc_pool(parts3, batch2, W3_0, Wout, bout2, N):
    B = 2000
    NB = N // B
    W3 = parts3.shape[-1]

    def body(p_ref, b_ref, w3_ref, wo_ref, bo_ref, o_ref, g_acc):
        i = pl.program_id(0)

        @pl.when(i == 0)
        def _():
            g_acc[...] = jnp.zeros_like(g_acc)

        ph = (p_ref[0] + p_ref[1])[:, :4]
        h3 = jnp.dot(ph, w3_ref[...], preferred_element_type=jnp.float32)
        h3 = h3 * jax.nn.sigmoid(h3)
        bb = b_ref[...]
        oneh = (bb == lax.broadcasted_iota(jnp.int32, (B, G), 1)
                ).astype(jnp.float32)
        g_acc[...] += lax.dot_general(
            oneh, h3, dimension_numbers=(((0,), (0,)), ((), ())),
            preferred_element_type=jnp.float32)

        @pl.when(i == NB - 1)
        def _():
            g = jnp.dot(g_acc[...], wo_ref[...],
                        preferred_element_type=jnp.float32) + bo_ref[...]
            o_ref[...] = jax.nn.softmax(g, axis=-1)

    return pl.pallas_call(
        body,
        grid=(NB,),
        in_specs=[
            pl.BlockSpec((2, B, W3), lambda i: (0, i, 0)),
            pl.BlockSpec((B, 1), lambda i: (i, 0)),
            pl.BlockSpec((4, 4), lambda i: (0, 0)),
            pl.BlockSpec((4, 8), lambda i: (0, 0)),
            pl.BlockSpec((1, 8), lambda i: (0, 0)),
        ],
        out_specs=pl.BlockSpec((G, 8), lambda i: (0, 0)),
        out_shape=jax.ShapeDtypeStruct((G, 8), jnp.float32),
        scratch_shapes=[pltpu.VMEM((G, H), jnp.float32)],
    )(parts3, batch2, W3_0, Wout, bout2)


def _degree_selector():
    s = np.zeros((16, 16), np.float32)
    for c in range(4):
        s[c, c] = 1.0
        for i in range(1, 4):
            for j in range(1, 4):
                s[i * 4 + c, j * 4 + c] = 1.0
    return jnp.asarray(s)


def kernel(x, edge_index, D1, edge_len, batch, wp1_W, wp1_b, W1_0, W1_1,
           wp2_W, wp2_b, W2_0, W2_1, wp3_W, wp3_b, W3_0, Wout, bout):
    N = x.shape[0]
    E = edge_len.shape[0]
    src2 = edge_index[0].astype(jnp.int32)
    dst2 = edge_index[1].astype(jnp.int32)
    d1r = D1.reshape(E, 16)
    x1 = x.reshape(N)
    S = _degree_selector()
    from jax.scipy.linalg import block_diag
    M1 = block_diag(W1_0, W1_1, W1_1, W1_1)
    M2 = block_diag(W2_0, W2_1, W2_1, W2_1)

    def bc(w):
        return jnp.broadcast_to(w.reshape(-1, 1), (w.size, 16))

    z16 = jnp.zeros((N, 16), jnp.float32)
    z8 = jnp.zeros((N, 8), jnp.float32)
    p1 = _sc_layer1(x1, src2, dst2, d1r, edge_len,
                    bc(wp1_W), bc(wp1_b), z16, N, E)
    h = _tc_node(p1.reshape(2, N, 16), M1, S, N)
    p2 = _sc_layer2(h, src2, dst2, d1r, edge_len,
                    bc(wp2_W), bc(wp2_b), z16, N, E)
    h = _tc_node(p2.reshape(2, N, 16), M2, S, N)
    p3 = _sc_layer3(h, src2, dst2, d1r, edge_len,
                    bc(wp3_W), bc(wp3_b), z8, N, E)
    out = _tc_pool(p3.reshape(2, N, p3.shape[-1]),
                   batch.astype(jnp.int32).reshape(N, 1),
                   W3_0, Wout, bout.reshape(1, 8), N)
    return out


# speedup vs baseline: 117.6307x; 1.3904x over previous
import functools

import jax
import jax.numpy as jnp
import numpy as np
from jax import lax
from jax.experimental import pallas as pl
from jax.experimental.pallas import tpu as pltpu
from jax.experimental.pallas import tpu_sc as plsc

H = 4
G = 64
NCORES = 2
NSUB = 16
NW = NCORES * NSUB
K = 400
SUB = 80
NSB = K // SUB
MU = [float(v) for v in np.linspace(0.7, 1.7, 10)]
LOG2 = float(np.log(2.0))

_SC_PARAMS = dict(
    compiler_params=pltpu.CompilerParams(
        needs_layout_passes=False, use_tc_tiling_on_sc=False))


def _iota16():
    return lax.broadcasted_iota(jnp.int32, (16,), 0)


def _emb_vecs(el):
    return [jnp.exp((el - m) * (el - m) * -50.0) for m in MU]


def _cutoff(el):
    u = el * (1.0 / 1.5)
    u2 = u * u
    u4 = u2 * u2
    u6 = u4 * u2
    u7 = u6 * u
    u8 = u4 * u4
    cut = 1.0 - 28.0 * u6 + 48.0 * u7 - 21.0 * u8
    return jnp.where(u < 1.0, cut, jnp.zeros_like(cut))


def _lw_vecs(embs, wp_ref, wpb_ref, nout):
    out = []
    for o in range(nout):
        v = embs[0] * wp_ref[0 * nout + o, :]
        for k in range(1, 10):
            v = v + embs[k] * wp_ref[k * nout + o, :]
        out.append(v + wpb_ref[o, :])
    return out


def _edge_bufs(width):
    return [
        pltpu.VMEM((K,), jnp.int32),
        pltpu.VMEM((NSB, SUB), jnp.int32),
        pltpu.VMEM((K, 16), jnp.float32),
        pltpu.VMEM((K,), jnp.float32),
        pltpu.VMEM((K, 16), jnp.float32),
        pltpu.VMEM((K, width), jnp.float32),
    ]


def _load_copies(src_hbm, dst_hbm, d1_hbm, el_hbm, bufs, off, sem):
    srcv, dstv, d1v, elv = bufs[0], bufs[1], bufs[2], bufs[3]
    ops = [
        (src_hbm.at[pl.ds(off, K)], srcv),
        (d1_hbm.at[pl.ds(off, K)], d1v),
        (el_hbm.at[pl.ds(off, K)], elv),
    ]
    for s0 in range(NSB):
        ops.append((dst_hbm.at[pl.ds(off + s0 * SUB, SUB)], dstv.at[s0]))
    return [pltpu.make_async_copy(a, b, sem) for a, b in ops]


def _gather_copies(h_hbm, bufs, sem):
    srcv, hrows = bufs[0], bufs[4]
    return [pltpu.make_async_copy(
        h_hbm.at[srcv.at[pl.ds(s * SUB, SUB)]],
        hrows.at[pl.ds(s * SUB, SUB)], sem) for s in range(NSB)]


def _scatter_copies(acc, bufs, sem):
    dstv, mrows = bufs[1], bufs[5]
    return [pltpu.make_async_copy(
        mrows.at[pl.ds(s * SUB, SUB)], acc.at[dstv.at[s]], sem)
        for s in range(NSB)]


def _pipeline(NCH, fire_loads, stage_gather, stage_compute, wait_scatter):
    fire_loads(0, 0)
    stage_gather(0, 0)

    def it(c, carry):
        for par in (0, 1):
            opar = 1 - par

            @pl.when(c % 2 == par)
            def _(par=par, opar=opar, c=c):
                @pl.when(c + 1 < NCH)
                def _():
                    fire_loads(opar, c + 1)
                stage_compute(par, c)

                @pl.when(c + 1 < NCH)
                def _():
                    stage_gather(opar, c + 1)
                wait_scatter(par)
        return carry
    lax.fori_loop(0, NCH, it, 0)


def _sc_layer1(x1, src2, dst2, d1r, elen, wpW, wpb, z16, N, E):
    NCH = (E // NW) // K
    mesh = plsc.VectorSubcoreMesh(core_axis_name="c", subcore_axis_name="s")

    @functools.partial(
        pl.kernel,
        out_type=jax.ShapeDtypeStruct((2 * N, 16), jnp.float32),
        mesh=mesh,
        scratch_types=(
            [pltpu.VMEM((N,), jnp.float32)]
            + _edge_bufs(16) + _edge_bufs(16)
            + [pltpu.VMEM((80, 16), jnp.float32),
               pltpu.VMEM((8, 16), jnp.float32),
               pltpu.VMEM_SHARED((N, 16), jnp.float32)]
            + [pltpu.SemaphoreType.DMA] * 4),
        **_SC_PARAMS,
    )
    def k(x_hbm, src_hbm, dst_hbm, d1_hbm, el_hbm, wpW_hbm, wpb_hbm, z_hbm,
          out_hbm,
          xv, a0, a1, a2, a3, a4, a5, b0, b1, b2, b3, b4, b5,
          wpv, wpbv, acc, sl0, sl1, ss0, ss1):
        cid = lax.axis_index("c")
        sid = lax.axis_index("s")
        wid = sid * NCORES + cid
        bufs = [[a0, a1, a2, a3, a4, a5], [b0, b1, b2, b3, b4, b5]]
        sls = [sl0, sl1]
        sss = [ss0, ss1]

        @pl.when(sid == 0)
        def _zero_acc():
            pltpu.sync_copy(z_hbm, acc)
        pltpu.sync_copy(x_hbm, xv)
        pltpu.sync_copy(wpW_hbm, wpv)
        pltpu.sync_copy(wpb_hbm, wpbv)
        plsc.subcore_barrier()

        ebase = wid * (E // NW)

        def fire_loads(par, c):
            off = ebase + c * K
            for cp in _load_copies(src_hbm, dst_hbm, d1_hbm, el_hbm,
                                   bufs[par], off, sls[par]):
                cp.start()

        def stage_gather(par, c):
            off = ebase + c * K
            for cp in _load_copies(src_hbm, dst_hbm, d1_hbm, el_hbm,
                                   bufs[par], off, sls[par]):
                cp.wait()

        def stage_compute(par, c):
            srcv, dstv, d1v, elv, hrows, mrows = bufs[par]

            def grp(g, c3):
                b = g * 16
                rows16 = _iota16() + b
                el = elv[pl.ds(b, 16)]
                src16 = srcv[pl.ds(b, 16)]
                xs = plsc.load_gather(xv, [src16])
                d0 = [plsc.load_gather(
                    d1v, [rows16, jnp.full((16,), j, jnp.int32)])
                    for j in range(4)]
                d2 = [plsc.load_gather(
                    d1v, [rows16, jnp.full((16,), 8 + j, jnp.int32)])
                    for j in range(4)]
                embs = _emb_vecs(el)
                lw = _lw_vecs(embs, wpv, wpbv, 8)
                cut = _cutoff(el)
                xr = d0[0] * xs
                y0 = [xr * lw[c2] for c2 in range(H)]
                y2 = [xr * lw[4 + c2] for c2 in range(H)]
                for i in range(4):
                    for c2 in range(H):
                        o = (d0[i] * y0[c2] + d2[i] * y2[c2]) * cut
                        plsc.store_scatter(
                            mrows,
                            [rows16, jnp.full((16,), i * 4 + c2, jnp.int32)],
                            o)
                return c3
            lax.fori_loop(0, K // 16, grp, 0)
            for cp in _scatter_copies(acc, bufs[par], sss[par]):
                cp.start(add=True)

        def wait_scatter(par):
            for cp in _scatter_copies(acc, bufs[par], sss[par]):
                cp.wait()

        _pipeline(NCH, fire_loads, stage_gather, stage_compute, wait_scatter)
        plsc.subcore_barrier()

        @pl.when(sid == 0)
        def _readout():
            pltpu.sync_copy(acc, out_hbm.at[pl.ds(cid * N, N)])

    return k(x1, src2, dst2, d1r, elen, wpW, wpb, z16)


def _sc_layer2(h, src2, dst2, d1r, elen, wpW, wpb, z16, N, E):
    NCH = (E // NW) // K
    mesh = plsc.VectorSubcoreMesh(core_axis_name="c", subcore_axis_name="s")

    @functools.partial(
        pl.kernel,
        out_type=jax.ShapeDtypeStruct((2 * N, 16), jnp.float32),
        mesh=mesh,
        scratch_types=(
            _edge_bufs(16) + _edge_bufs(16)
            + [pltpu.VMEM((240, 16), jnp.float32),
               pltpu.VMEM((24, 16), jnp.float32),
               pltpu.VMEM_SHARED((N, 16), jnp.float32)]
            + [pltpu.SemaphoreType.DMA] * 6),
        **_SC_PARAMS,
    )
    def k(h_hbm, src_hbm, dst_hbm, d1_hbm, el_hbm, wpW_hbm, wpb_hbm, z_hbm,
          out_hbm,
          a0, a1, a2, a3, a4, a5, b0, b1, b2, b3, b4, b5,
          wpv, wpbv, acc, sl0, sl1, sg0, sg1, ss0, ss1):
        cid = lax.axis_index("c")
        sid = lax.axis_index("s")
        wid = sid * NCORES + cid
        bufs = [[a0, a1, a2, a3, a4, a5], [b0, b1, b2, b3, b4, b5]]
        sls = [sl0, sl1]
        sgs = [sg0, sg1]
        sss = [ss0, ss1]

        @pl.when(sid == 0)
        def _zero_acc():
            pltpu.sync_copy(z_hbm, acc)
        pltpu.sync_copy(wpW_hbm, wpv)
        pltpu.sync_copy(wpb_hbm, wpbv)
        plsc.subcore_barrier()

        ebase = wid * (E // NW)

        def fire_loads(par, c):
            off = ebase + c * K
            for cp in _load_copies(src_hbm, dst_hbm, d1_hbm, el_hbm,
                                   bufs[par], off, sls[par]):
                cp.start()

        def stage_gather(par, c):
            off = ebase + c * K
            for cp in _load_copies(src_hbm, dst_hbm, d1_hbm, el_hbm,
                                   bufs[par], off, sls[par]):
                cp.wait()
            for cp in _gather_copies(h_hbm, bufs[par], sgs[par]):
                cp.start()

        def stage_compute(par, c):
            srcv, dstv, d1v, elv, hrows, mrows = bufs[par]
            for cp in _gather_copies(h_hbm, bufs[par], sgs[par]):
                cp.wait()

            def grp(g, c3):
                b = g * 16
                rows16 = _iota16() + b
                el = elv[pl.ds(b, 16)]
                d = [plsc.load_gather(
                    d1v, [rows16, jnp.full((16,), j, jnp.int32)])
                    for j in range(16)]
                hs = [plsc.load_gather(
                    hrows, [rows16, jnp.full((16,), j, jnp.int32)])
                    for j in range(16)]
                embs = _emb_vecs(el)
                lw = _lw_vecs(embs, wpv, wpbv, 24)
                cut = _cutoff(el)
                xj = [[None] * H for _ in range(4)]
                for i in range(4):
                    for c2 in range(H):
                        v = d[i * 4] * hs[c2]
                        for j in range(1, 4):
                            v = v + d[i * 4 + j] * hs[j * 4 + c2]
                        xj[i][c2] = v
                m = [[None] * H for _ in range(4)]
                for c2 in range(H):
                    m[0][c2] = lw[c2] * xj[0][c2] + lw[8 + c2] * xj[2][c2]
                    m[2][c2] = lw[4 + c2] * xj[0][c2] + lw[12 + c2] * xj[2][c2]
                    m[3][c2] = lw[16 + c2] * xj[3][c2] - lw[20 + c2] * xj[1][c2]
                    m[1][c2] = lw[20 + c2] * xj[3][c2] + lw[16 + c2] * xj[1][c2]
                for i in range(4):
                    for c2 in range(H):
                        v = d[i] * m[0][c2]
                        for j in range(1, 4):
                            v = v + d[j * 4 + i] * m[j][c2]
                        plsc.store_scatter(
                            mrows,
                            [rows16, jnp.full((16,), i * 4 + c2, jnp.int32)],
                            v * cut)
                return c3
            lax.fori_loop(0, K // 16, grp, 0)
            for cp in _scatter_copies(acc, bufs[par], sss[par]):
                cp.start(add=True)

        def wait_scatter(par):
            for cp in _scatter_copies(acc, bufs[par], sss[par]):
                cp.wait()

        _pipeline(NCH, fire_loads, stage_gather, stage_compute, wait_scatter)
        plsc.subcore_barrier()

        @pl.when(sid == 0)
        def _readout():
            pltpu.sync_copy(acc, out_hbm.at[pl.ds(cid * N, N)])

    return k(h, src2, dst2, d1r, elen, wpW, wpb, z16)


def _sc_layer3(h, src2, dst2, d1r, elen, wpW, wpb, z8, N, E):
    NCH = (E // NW) // K
    W3 = 8
    mesh = plsc.VectorSubcoreMesh(core_axis_name="c", subcore_axis_name="s")

    @functools.partial(
        pl.kernel,
        out_type=jax.ShapeDtypeStruct((2 * N, W3), jnp.float32),
        mesh=mesh,
        scratch_types=(
            _edge_bufs(W3) + _edge_bufs(W3)
            + [pltpu.VMEM((80, 16), jnp.float32),
               pltpu.VMEM((8, 16), jnp.float32),
               pltpu.VMEM_SHARED((N, W3), jnp.float32)]
            + [pltpu.SemaphoreType.DMA] * 6),
        **_SC_PARAMS,
    )
    def k(h_hbm, src_hbm, dst_hbm, d1_hbm, el_hbm, wpW_hbm, wpb_hbm, z_hbm,
          out_hbm,
          a0, a1, a2, a3, a4, a5, b0, b1, b2, b3, b4, b5,
          wpv, wpbv, acc, sl0, sl1, sg0, sg1, ss0, ss1):
        cid = lax.axis_index("c")
        sid = lax.axis_index("s")
        wid = sid * NCORES + cid
        bufs = [[a0, a1, a2, a3, a4, a5], [b0, b1, b2, b3, b4, b5]]
        sls = [sl0, sl1]
        sgs = [sg0, sg1]
        sss = [ss0, ss1]

        iota = _iota16()
        zr = lax.shift_right_logical(iota, 3)
        zc = lax.bitwise_and(iota, 7)
        zv = jnp.zeros((16,), jnp.float32)

        def zq(i, c):
            plsc.store_scatter(a5, [i * 2 + zr, zc], zv)
            plsc.store_scatter(b5, [i * 2 + zr, zc], zv)
            return c
        lax.fori_loop(0, K // 2, zq, 0)

        @pl.when(sid == 0)
        def _zero_acc():
            pltpu.sync_copy(z_hbm, acc)
        pltpu.sync_copy(wpW_hbm, wpv)
        pltpu.sync_copy(wpb_hbm, wpbv)
        plsc.subcore_barrier()

        ebase = wid * (E // NW)

        def fire_loads(par, c):
            off = ebase + c * K
            for cp in _load_copies(src_hbm, dst_hbm, d1_hbm, el_hbm,
                                   bufs[par], off, sls[par]):
                cp.start()

        def stage_gather(par, c):
            off = ebase + c * K
            for cp in _load_copies(src_hbm, dst_hbm, d1_hbm, el_hbm,
                                   bufs[par], off, sls[par]):
                cp.wait()
            for cp in _gather_copies(h_hbm, bufs[par], sgs[par]):
                cp.start()

        def stage_compute(par, c):
            srcv, dstv, d1v, elv, hrows, mrows = bufs[par]
            for cp in _gather_copies(h_hbm, bufs[par], sgs[par]):
                cp.wait()

            def grp(g, c3):
                b = g * 16
                rows16 = _iota16() + b
                el = elv[pl.ds(b, 16)]
                d0 = [plsc.load_gather(
                    d1v, [rows16, jnp.full((16,), j, jnp.int32)])
                    for j in range(4)]
                d2 = [plsc.load_gather(
                    d1v, [rows16, jnp.full((16,), 8 + j, jnp.int32)])
                    for j in range(4)]
                hs = [plsc.load_gather(
                    hrows, [rows16, jnp.full((16,), j, jnp.int32)])
                    for j in range(16)]
                embs = _emb_vecs(el)
                lw = _lw_vecs(embs, wpv, wpbv, 8)
                cut = _cutoff(el)
                dc = d0[0] * cut
                for c2 in range(H):
                    xj0 = d0[0] * hs[c2]
                    xj2 = d2[0] * hs[c2]
                    for j in range(1, 4):
                        xj0 = xj0 + d0[j] * hs[j * 4 + c2]
                        xj2 = xj2 + d2[j] * hs[j * 4 + c2]
                    y0 = lw[c2] * xj0 + lw[4 + c2] * xj2
                    plsc.store_scatter(
                        mrows, [rows16, jnp.full((16,), c2, jnp.int32)],
                        y0 * dc)
                return c3
            lax.fori_loop(0, K // 16, grp, 0)
            for cp in _scatter_copies(acc, bufs[par], sss[par]):
                cp.start(add=True)

        def wait_scatter(par):
            for cp in _scatter_copies(acc, bufs[par], sss[par]):
                cp.wait()

        _pipeline(NCH, fire_loads, stage_gather, stage_compute, wait_scatter)
        plsc.subcore_barrier()

        @pl.when(sid == 0)
        def _readout():
            pltpu.sync_copy(acc, out_hbm.at[pl.ds(cid * N, N)])

    return k(h, src2, dst2, d1r, elen, wpW, wpb, z8)


def _tc_node(parts, M, S, N):
    B = 2000
    NB = N // B

    def body(p_ref, m_ref, s_ref, o_ref):
        ph = p_ref[0] + p_ref[1]
        h2 = jnp.dot(ph, m_ref[...], preferred_element_type=jnp.float32)
        nsq = jnp.dot(h2 * h2, s_ref[...], preferred_element_type=jnp.float32)
        n = jnp.sqrt(nsq + 1e-12)
        act = jax.nn.softplus(n) - LOG2
        o_ref[...] = h2 * (act / n)

    return pl.pallas_call(
        body,
        grid=(NB,),
        in_specs=[
            pl.BlockSpec((2, B, 16), lambda i: (0, i, 0)),
            pl.BlockSpec((16, 16), lambda i: (0, 0)),
            pl.BlockSpec((16, 16), lambda i: (0, 0)),
        ],
        out_specs=pl.BlockSpec((B, 16), lambda i: (i, 0)),
        out_shape=jax.ShapeDtypeStruct((N, 16), jnp.float32),
    )(parts, M, S)


def _tc_pool(parts3, batch2, W3_0, Wout, bout2, N):
    B = 2000
    NB = N // B
    W3 = parts3.shape[-1]

    def body(p_ref, b_ref, w3_ref, wo_ref, bo_ref, o_ref, g_acc):
        i = pl.program_id(0)

        @pl.when(i == 0)
        def _():
            g_acc[...] = jnp.zeros_like(g_acc)

        ph = (p_ref[0] + p_ref[1])[:, :4]
        h3 = jnp.dot(ph, w3_ref[...], preferred_element_type=jnp.float32)
        h3 = h3 * jax.nn.sigmoid(h3)
        bb = b_ref[...]
        oneh = (bb == lax.broadcasted_iota(jnp.int32, (B, G), 1)
                ).astype(jnp.float32)
        g_acc[...] += lax.dot_general(
            oneh, h3, dimension_numbers=(((0,), (0,)), ((), ())),
            preferred_element_type=jnp.float32)

        @pl.when(i == NB - 1)
        def _():
            g = jnp.dot(g_acc[...], wo_ref[...],
                        preferred_element_type=jnp.float32) + bo_ref[...]
            o_ref[...] = jax.nn.softmax(g, axis=-1)

    return pl.pallas_call(
        body,
        grid=(NB,),
        in_specs=[
            pl.BlockSpec((2, B, W3), lambda i: (0, i, 0)),
            pl.BlockSpec((B, 1), lambda i: (i, 0)),
            pl.BlockSpec((4, 4), lambda i: (0, 0)),
            pl.BlockSpec((4, 8), lambda i: (0, 0)),
            pl.BlockSpec((1, 8), lambda i: (0, 0)),
        ],
        out_specs=pl.BlockSpec((G, 8), lambda i: (0, 0)),
        out_shape=jax.ShapeDtypeStruct((G, 8), jnp.float32),
        scratch_shapes=[pltpu.VMEM((G, H), jnp.float32)],
    )(parts3, batch2, W3_0, Wout, bout2)


def _degree_selector():
    s = np.zeros((16, 16), np.float32)
    for c in range(4):
        s[c, c] = 1.0
        for i in range(1, 4):
            for j in range(1, 4):
                s[i * 4 + c, j * 4 + c] = 1.0
    return jnp.asarray(s)


def kernel(x, edge_index, D1, edge_len, batch, wp1_W, wp1_b, W1_0, W1_1,
           wp2_W, wp2_b, W2_0, W2_1, wp3_W, wp3_b, W3_0, Wout, bout):
    N = x.shape[0]
    E = edge_len.shape[0]
    src2 = edge_index[0].astype(jnp.int32)
    dst2 = edge_index[1].astype(jnp.int32)
    d1r = D1.reshape(E, 16)
    x1 = x.reshape(N)
    S = _degree_selector()
    from jax.scipy.linalg import block_diag
    M1 = block_diag(W1_0, W1_1, W1_1, W1_1)
    M2 = block_diag(W2_0, W2_1, W2_1, W2_1)

    def bc(w):
        return jnp.broadcast_to(w.reshape(-1, 1), (w.size, 16))

    z16 = jnp.zeros((N, 16), jnp.float32)
    z8 = jnp.zeros((N, 8), jnp.float32)
    p1 = _sc_layer1(x1, src2, dst2, d1r, edge_len,
                    bc(wp1_W), bc(wp1_b), z16, N, E)
    h = _tc_node(p1.reshape(2, N, 16), M1, S, N)
    p2 = _sc_layer2(h, src2, dst2, d1r, edge_len,
                    bc(wp2_W), bc(wp2_b), z16, N, E)
    h = _tc_node(p2.reshape(2, N, 16), M2, S, N)
    p3 = _sc_layer3(h, src2, dst2, d1r, edge_len,
                    bc(wp3_W), bc(wp3_b), z8, N, E)
    out = _tc_pool(p3.reshape(2, N, p3.shape[-1]),
                   batch.astype(jnp.int32).reshape(N, 1),
                   W3_0, Wout, bout.reshape(1, 8), N)
    return out
